# Initial kernel scaffold; baseline (speedup 1.0000x reference)
import jax
import jax.numpy as jnp
from jax.experimental import pallas as pl


def kernel(inputs, targets, table):
    raise NotImplementedError("write your pallas kernel here")


# speedup vs baseline: 1.8842x; 1.8842x over previous
import jax
import jax.numpy as jnp
from jax import lax
from jax.experimental import pallas as pl
from jax.experimental.pallas import tpu as pltpu
from jax.experimental.pallas import tpu_sc as plsc

VOCAB = 8192
NC, NS = 2, 16
NW = NC * NS
K = 4
ROWS_PER_W = 128
NCHUNK = ROWS_PER_W // K


def _body(idx_hbm, table_hbm, out_hbm, idx_v, buf_v, g0, g1, o0, o1):
    wid = lax.axis_index("s") * NC + lax.axis_index("c")
    row0 = wid * ROWS_PER_W

    pltpu.sync_copy(idx_hbm.at[wid], idx_v)

    gsem = (g0, g1)
    osem = (o0, o1)

    def g_start(c, b):
        pltpu.make_async_copy(
            table_hbm.at[idx_v.at[c]], buf_v.at[b], gsem[b]).start()

    def g_wait(b):
        pltpu.make_async_copy(
            table_hbm.at[idx_v.at[0]], buf_v.at[b], gsem[b]).wait()

    def o_start(c, b):
        pltpu.make_async_copy(
            buf_v.at[b], out_hbm.at[pl.ds(row0 + c * K, K)], osem[b]).start()

    def o_wait(b):
        pltpu.make_async_copy(
            buf_v.at[b], out_hbm.at[pl.ds(row0, K)], osem[b]).wait()

    g_start(0, 0)
    g_start(1, 1)
    g_wait(0)
    o_start(0, 0)

    def loop_body(i, _):
        c = 2 * i + 1
        o_wait(0)
        g_start(c + 1, 0)
        g_wait(1)
        o_start(c, 1)

        c = 2 * i + 2
        o_wait(1)
        g_start(c + 1, 1)
        g_wait(0)
        o_start(c, 0)
        return _

    lax.fori_loop(0, (NCHUNK - 2) // 2, loop_body, None)

    o_wait(0)
    g_wait(1)
    o_start(NCHUNK - 1, 1)
    o_wait(1)


_gather = pl.kernel(
    _body,
    out_type=jax.ShapeDtypeStruct((NW * ROWS_PER_W, VOCAB), jnp.float32),
    mesh=plsc.VectorSubcoreMesh(core_axis_name="c", subcore_axis_name="s"),
    scratch_types=[
        pltpu.VMEM((NCHUNK, K), jnp.int32),
        pltpu.VMEM((2, K, VOCAB), jnp.float32),
        pltpu.SemaphoreType.DMA,
        pltpu.SemaphoreType.DMA,
        pltpu.SemaphoreType.DMA,
        pltpu.SemaphoreType.DMA,
    ],
)


def kernel(inputs, targets, table):
    del targets
    b, l = inputs.shape
    idx = inputs.astype(jnp.int32).reshape(NW, NCHUNK, K)
    out = _gather(idx, table)
    return out.reshape(b, l, VOCAB)
